# Initial kernel scaffold; baseline (speedup 1.0000x reference)
import jax
import jax.numpy as jnp
from jax.experimental import pallas as pl


def kernel(xyz, features, inds, W1, b1, W2, b2):
    raise NotImplementedError("write your pallas kernel here")


# speedup vs baseline: 15.0937x; 15.0937x over previous
import numpy as np
import jax
import jax.numpy as jnp
from jax.experimental import pallas as pl

_NS = 32
_R2 = np.float32(0.2 ** 2)


def _ball_kernel(xt_ref, c_ref, o_ref):
    xt = xt_ref[0]
    c = c_ref[0]
    n = xt.shape[1]
    d2 = None
    for d in range(3):
        dx = c[:, d:d + 1] - xt[d:d + 1, :]
        d2 = dx * dx if d2 is None else d2 + dx * dx
    iota = jax.lax.broadcasted_iota(jnp.int32, d2.shape, 1)
    keys = jnp.where(d2 < _R2, iota, n)
    cols = []
    for _ in range(_NS):
        m = jnp.min(keys, axis=1, keepdims=True)
        cols.append(m)
        keys = jnp.where(keys == m, 2 * n, keys)
    outk = jnp.concatenate(cols, axis=1)
    valid = outk < n
    first = jnp.where(valid[:, 0:1], outk[:, 0:1], 0)
    o_ref[0] = jnp.where(valid, outk, first)


def _mlp_kernel(gx_ref, gf_ref, c_ref, w1a_ref, w1b_ref, b1_ref,
                w2_ref, b2_ref, o_ref):
    c = c_ref[0]
    w1a = w1a_ref[...]
    w1b = w1b_ref[...]
    b1 = b1_ref[...]
    w2 = w2_ref[...]
    b2 = b2_ref[...]
    acc = None
    for j in range(_NS):
        dx = gx_ref[0, j] - c
        f = gf_ref[0, j]
        h = (jnp.dot(dx, w1a, preferred_element_type=jnp.float32)
             + jnp.dot(f, w1b, preferred_element_type=jnp.float32) + b1)
        h = jnp.maximum(h, 0.0)
        h2 = jnp.maximum(
            jnp.dot(h, w2, preferred_element_type=jnp.float32) + b2, 0.0)
        acc = h2 if acc is None else jnp.maximum(acc, h2)
    o_ref[0] = acc


def kernel(xyz, features, inds, W1, b1, W2, b2):
    B, N, _ = xyz.shape
    S = inds.shape[1]
    C = features.shape[1]
    O = W2.shape[0]

    inds = inds.astype(jnp.int32)
    new_xyz = jnp.take_along_axis(xyz, inds[:, :, None], axis=1)
    xyzT = jnp.transpose(xyz, (0, 2, 1))

    Sb = 128
    idx = pl.pallas_call(
        _ball_kernel,
        grid=(B, S // Sb),
        in_specs=[
            pl.BlockSpec((1, 3, N), lambda b, s: (b, 0, 0)),
            pl.BlockSpec((1, Sb, 3), lambda b, s: (b, s, 0)),
        ],
        out_specs=pl.BlockSpec((1, Sb, _NS), lambda b, s: (b, s, 0)),
        out_shape=jax.ShapeDtypeStruct((B, S, _NS), jnp.int32),
    )(xyzT, new_xyz)

    idxT = jnp.transpose(idx, (0, 2, 1))
    gx = jax.vmap(lambda pts, ii: pts[ii])(xyz, idxT)
    feats_t = jnp.transpose(features, (0, 2, 1))
    gf = jax.vmap(lambda f, ii: f[ii])(feats_t, idxT)

    w1a = jnp.transpose(W1[:, :3])
    w1b = jnp.transpose(W1[:, 3:])
    w2t = jnp.transpose(W2)
    b1r = b1[None, :]
    b2r = b2[None, :]

    Sm = 256
    pooled = pl.pallas_call(
        _mlp_kernel,
        grid=(B, S // Sm),
        in_specs=[
            pl.BlockSpec((1, _NS, Sm, 3), lambda b, s: (b, 0, s, 0)),
            pl.BlockSpec((1, _NS, Sm, C), lambda b, s: (b, 0, s, 0)),
            pl.BlockSpec((1, Sm, 3), lambda b, s: (b, s, 0)),
            pl.BlockSpec((3, 64), lambda b, s: (0, 0)),
            pl.BlockSpec((C, 64), lambda b, s: (0, 0)),
            pl.BlockSpec((1, 64), lambda b, s: (0, 0)),
            pl.BlockSpec((64, O), lambda b, s: (0, 0)),
            pl.BlockSpec((1, O), lambda b, s: (0, 0)),
        ],
        out_specs=pl.BlockSpec((1, Sm, O), lambda b, s: (b, s, 0)),
        out_shape=jax.ShapeDtypeStruct((B, S, O), jnp.float32),
    )(gx, gf, new_xyz, w1a, w1b, b1r, w2t, b2r)

    new_features = jnp.transpose(pooled, (0, 2, 1))
    return (new_xyz, new_features, inds)
